# Initial kernel scaffold; baseline (speedup 1.0000x reference)
import jax
import jax.numpy as jnp
from jax.experimental import pallas as pl


def kernel(x, edge_index, W_l1, W_r1, b1, W_l2, W_r2, b2):
    raise NotImplementedError("write your pallas kernel here")


# speedup vs baseline: 4.6483x; 4.6483x over previous
import functools

import jax
import jax.numpy as jnp
from jax import lax
from jax.experimental import pallas as pl
from jax.experimental.pallas import tpu as pltpu
from jax.experimental.pallas import tpu_sc as plsc

_N = 10000
_E = 320000
_IDX_ROWS = 2560
_EP = _IDX_ROWS * 128
_TBL = 10240
_PAD_DST = _N
_NS = 16
_F32 = jnp.float32


def _make_sc_agg(width, with_cnt):
    rows_sub = _TBL // _NS
    idx_sub = _IDX_ROWS // _NS
    cnt_sub = _IDX_ROWS // (2 * _NS)

    def body(*refs):
        if with_cnt:
            (xlo, xhi, srcp, dstp, ones_h,
             olo, ohi, cnt_a, cnt_b,
             src_v, dst_v, dstc_v, rows_v, ones_v, zbuf, zbuf_c,
             agg_sp, cnt_sp, sem) = refs
        else:
            (xlo, xhi, srcp, dstp,
             olo, ohi,
             src_v, dst_v, rows_v, zbuf,
             agg_sp, sem) = refs
        c = lax.axis_index("c")
        s = lax.axis_index("s")

        for r in range(16):
            for cc in range(width // 16):
                zbuf[r, pl.ds(cc * 16, 16)] = jnp.zeros((16,), _F32)
        row0 = s * rows_sub

        @pl.loop(0, rows_sub // 16)
        def _(k):
            pltpu.sync_copy(zbuf, agg_sp.at[pl.ds(row0 + k * 16, 16), :])

        if with_cnt:
            for r in range(16):
                zbuf_c[r, :] = jnp.zeros((16,), _F32)

            @pl.loop(0, rows_sub // 16)
            def _(k):
                pltpu.sync_copy(zbuf_c, cnt_sp.at[pl.ds(row0 + k * 16, 16), :])

            pltpu.sync_copy(ones_h, ones_v)

        pltpu.sync_copy(srcp.at[pl.ds(s * idx_sub, idx_sub), :], src_v)
        pltpu.sync_copy(dstp.at[pl.ds(s * idx_sub, idx_sub), :], dst_v)
        if with_cnt:
            cbase = c * (_IDX_ROWS // 2) + s * cnt_sub
            pltpu.sync_copy(dstp.at[pl.ds(cbase, cnt_sub), :], dstc_v)
        plsc.subcore_barrier()

        def edge_loop(x_hbm):
            @pl.loop(0, idx_sub)
            def _(j):
                pltpu.async_copy(x_hbm.at[src_v.at[j]], rows_v, sem).wait()
                pltpu.sync_copy(rows_v, agg_sp.at[dst_v.at[j]], add=True)

        @pl.when(c == 0)
        def _():
            edge_loop(xlo)

        @pl.when(c == 1)
        def _():
            edge_loop(xhi)

        if with_cnt:
            @pl.loop(0, cnt_sub)
            def _(j):
                pltpu.sync_copy(ones_v, cnt_sp.at[dstc_v.at[j]], add=True)

        plsc.subcore_barrier()

        @pl.when(c == 0)
        def _():
            pltpu.sync_copy(agg_sp.at[pl.ds(row0, rows_sub), :],
                            olo.at[pl.ds(row0, rows_sub), :])
            if with_cnt:
                pltpu.sync_copy(cnt_sp.at[pl.ds(row0, rows_sub), :],
                                cnt_a.at[pl.ds(row0, rows_sub), :])

        @pl.when(c == 1)
        def _():
            pltpu.sync_copy(agg_sp.at[pl.ds(row0, rows_sub), :],
                            ohi.at[pl.ds(row0, rows_sub), :])
            if with_cnt:
                pltpu.sync_copy(cnt_sp.at[pl.ds(row0, rows_sub), :],
                                cnt_b.at[pl.ds(row0, rows_sub), :])

    out_type = [jax.ShapeDtypeStruct((_TBL, width), _F32),
                jax.ShapeDtypeStruct((_TBL, width), _F32)]
    scratch = [pltpu.VMEM((idx_sub, 128), jnp.int32),
               pltpu.VMEM((idx_sub, 128), jnp.int32)]
    if with_cnt:
        out_type += [jax.ShapeDtypeStruct((_TBL, 16), _F32),
                     jax.ShapeDtypeStruct((_TBL, 16), _F32)]
        scratch += [pltpu.VMEM((cnt_sub, 128), jnp.int32)]
    scratch += [pltpu.VMEM((128, width), _F32)]
    if with_cnt:
        scratch += [pltpu.VMEM((128, 16), _F32)]
    scratch += [pltpu.VMEM((16, width), _F32)]
    if with_cnt:
        scratch += [pltpu.VMEM((16, 16), _F32)]
    scratch += [pltpu.VMEM_SHARED((_TBL, width), _F32)]
    if with_cnt:
        scratch += [pltpu.VMEM_SHARED((_TBL, 16), _F32)]
    scratch += [pltpu.SemaphoreType.DMA]

    mesh = plsc.VectorSubcoreMesh(core_axis_name="c", subcore_axis_name="s")
    return pl.kernel(body, out_type=out_type, mesh=mesh,
                     scratch_types=scratch,
                     compiler_params=pltpu.CompilerParams(
                         use_tc_tiling_on_sc=False),
                     name=f"sc_segsum_w{width}")


_sc_agg64 = _make_sc_agg(64, with_cnt=True)
_sc_agg32 = _make_sc_agg(32, with_cnt=False)


def _tc1_body(alo, ahi, ca, cb, x, wl1, wr1, b1, wl2, wr2, b2,
              plo, phi, r2o):
    cnt = ca[:, 0:1] + cb[:, 0:1]
    icnt = 1.0 / jnp.maximum(cnt, 1.0)
    h = (jnp.dot(alo[...] * icnt, wl1[0:64, :], preferred_element_type=_F32)
         + jnp.dot(ahi[...] * icnt, wl1[64:128, :], preferred_element_type=_F32)
         + jnp.dot(x[...], wr1[...], preferred_element_type=_F32)
         + b1[...])
    h = jnp.maximum(h, 0.0)
    p = jnp.dot(h, wl2[...], preferred_element_type=_F32)
    plo[...] = p[:, 0:32]
    phi[...] = p[:, 32:64]
    r2o[...] = jnp.dot(h, wr2[...], preferred_element_type=_F32) + b2[...]


def _tc2_body(a2lo, a2hi, ca, cb, r2, out):
    cnt = ca[:, 0:1] + cb[:, 0:1]
    icnt = 1.0 / jnp.maximum(cnt, 1.0)
    out[...] = (jnp.concatenate([a2lo[...] * icnt, a2hi[...] * icnt], axis=1)
                + r2[...])


_BLK = 1000


def _row(i):
    return (i, 0)


def _full(i):
    return (0, 0)


def _tc1(alo, ahi, ca, cb, x, wl1, wr1, b1, wl2, wr2, b2):
    return pl.pallas_call(
        _tc1_body,
        grid=(_N // _BLK,),
        in_specs=[
            pl.BlockSpec((_BLK, 64), _row),
            pl.BlockSpec((_BLK, 64), _row),
            pl.BlockSpec((_BLK, 16), _row),
            pl.BlockSpec((_BLK, 16), _row),
            pl.BlockSpec((_BLK, 128), _row),
            pl.BlockSpec((128, 128), _full),
            pl.BlockSpec((128, 128), _full),
            pl.BlockSpec((1, 128), _full),
            pl.BlockSpec((128, 64), _full),
            pl.BlockSpec((128, 64), _full),
            pl.BlockSpec((1, 64), _full),
        ],
        out_specs=[
            pl.BlockSpec((_BLK, 32), _row),
            pl.BlockSpec((_BLK, 32), _row),
            pl.BlockSpec((_BLK, 64), _row),
        ],
        out_shape=[
            jax.ShapeDtypeStruct((_N, 32), _F32),
            jax.ShapeDtypeStruct((_N, 32), _F32),
            jax.ShapeDtypeStruct((_N, 64), _F32),
        ],
    )(alo, ahi, ca, cb, x, wl1, wr1, b1, wl2, wr2, b2)


def _tc2(a2lo, a2hi, ca, cb, r2):
    return pl.pallas_call(
        _tc2_body,
        grid=(_N // _BLK,),
        in_specs=[
            pl.BlockSpec((_BLK, 32), _row),
            pl.BlockSpec((_BLK, 32), _row),
            pl.BlockSpec((_BLK, 16), _row),
            pl.BlockSpec((_BLK, 16), _row),
            pl.BlockSpec((_BLK, 64), _row),
        ],
        out_specs=pl.BlockSpec((_BLK, 64), _row),
        out_shape=jax.ShapeDtypeStruct((_N, 64), _F32),
    )(a2lo, a2hi, ca, cb, r2)


def kernel(x, edge_index, W_l1, W_r1, b1, W_l2, W_r2, b2):
    src = edge_index[0].astype(jnp.int32)
    dst = edge_index[1].astype(jnp.int32)
    pad = _EP - _E
    srcp = jnp.concatenate([src, jnp.zeros((pad,), jnp.int32)]).reshape(
        _IDX_ROWS, 128)
    dstp = jnp.concatenate([dst, jnp.full((pad,), _PAD_DST, jnp.int32)]
                           ).reshape(_IDX_ROWS, 128)
    ones = jnp.ones((128, 16), _F32)
    x_lo = x[:, 0:64]
    x_hi = x[:, 64:128]

    agg_lo, agg_hi, cnt_a, cnt_b = _sc_agg64(x_lo, x_hi, srcp, dstp, ones)
    p_lo, p_hi, r2 = _tc1(agg_lo, agg_hi, cnt_a, cnt_b, x,
                          W_l1, W_r1, b1.reshape(1, -1),
                          W_l2, W_r2, b2.reshape(1, -1))
    a2_lo, a2_hi = _sc_agg32(p_lo, p_hi, srcp, dstp)
    return _tc2(a2_lo, a2_hi, cnt_a, cnt_b, r2)


# speedup vs baseline: 5.5742x; 1.1992x over previous
import functools

import jax
import jax.numpy as jnp
from jax import lax
from jax.experimental import pallas as pl
from jax.experimental.pallas import tpu as pltpu
from jax.experimental.pallas import tpu_sc as plsc

_N = 10000
_E = 320000
_IDX_ROWS = 2560
_EP = _IDX_ROWS * 128
_TBL = 10240
_PAD_DST = _N
_NS = 16
_NBUF = 4
_F32 = jnp.float32


def _make_sc_agg(width, with_cnt):
    rows_sub = _TBL // _NS
    idx_sub = _IDX_ROWS // _NS
    cnt_sub = _IDX_ROWS // (2 * _NS)

    def body(*refs):
        if with_cnt:
            (xlo, xhi, srcp, dstp, ones_h,
             olo, ohi, cnt_a, cnt_b,
             src_v, dst_v, dstc_v, rows_v, ones_v, zbuf, zbuf_c,
             agg_sp, cnt_sp, *sems) = refs
        else:
            (xlo, xhi, srcp, dstp,
             olo, ohi,
             src_v, dst_v, rows_v, zbuf,
             agg_sp, *sems) = refs
        gsem = sems[:_NBUF]
        ssem = sems[_NBUF:]
        c = lax.axis_index("c")
        s = lax.axis_index("s")

        for r in range(16):
            for cc in range(width // 16):
                zbuf[r, pl.ds(cc * 16, 16)] = jnp.zeros((16,), _F32)
        row0 = s * rows_sub

        @pl.loop(0, rows_sub // 16)
        def _(k):
            pltpu.sync_copy(zbuf, agg_sp.at[pl.ds(row0 + k * 16, 16), :])

        if with_cnt:
            for r in range(16):
                zbuf_c[r, :] = jnp.zeros((16,), _F32)

            @pl.loop(0, rows_sub // 16)
            def _(k):
                pltpu.sync_copy(zbuf_c, cnt_sp.at[pl.ds(row0 + k * 16, 16), :])

            pltpu.sync_copy(ones_h, ones_v)

        pltpu.sync_copy(srcp.at[pl.ds(s * idx_sub, idx_sub), :], src_v)
        pltpu.sync_copy(dstp.at[pl.ds(s * idx_sub, idx_sub), :], dst_v)
        plsc.subcore_barrier()

        def edge_loop(x_hbm):
            @pl.loop(0, idx_sub // _NBUF)
            def _(t):
                gds = []
                for b in range(_NBUF):
                    j = t * _NBUF + b
                    gds.append(pltpu.async_copy(
                        x_hbm.at[src_v.at[j]], rows_v.at[b], gsem[b]))
                sds = []
                for b in range(_NBUF):
                    j = t * _NBUF + b
                    gds[b].wait()
                    sds.append(pltpu.async_copy(
                        rows_v.at[b], agg_sp.at[dst_v.at[j]], ssem[b],
                        add=True))
                for sd in sds:
                    sd.wait()

        @pl.when(c == 0)
        def _():
            edge_loop(xlo)

        @pl.when(c == 1)
        def _():
            edge_loop(xhi)

        if with_cnt:
            cbase = c * (_IDX_ROWS // 2) + s * cnt_sub

            @pl.loop(0, cnt_sub // _NBUF)
            def _(t):
                pltpu.sync_copy(
                    dstp.at[pl.ds(cbase + t * _NBUF, _NBUF), :], dstc_v)
                sds = []
                for b in range(_NBUF):
                    sds.append(pltpu.async_copy(
                        ones_v, cnt_sp.at[dstc_v.at[b]], ssem[b], add=True))
                for sd in sds:
                    sd.wait()

        plsc.subcore_barrier()

        @pl.when(c == 0)
        def _():
            pltpu.sync_copy(agg_sp.at[pl.ds(row0, rows_sub), :],
                            olo.at[pl.ds(row0, rows_sub), :])
            if with_cnt:
                pltpu.sync_copy(cnt_sp.at[pl.ds(row0, rows_sub), :],
                                cnt_a.at[pl.ds(row0, rows_sub), :])

        @pl.when(c == 1)
        def _():
            pltpu.sync_copy(agg_sp.at[pl.ds(row0, rows_sub), :],
                            ohi.at[pl.ds(row0, rows_sub), :])
            if with_cnt:
                pltpu.sync_copy(cnt_sp.at[pl.ds(row0, rows_sub), :],
                                cnt_b.at[pl.ds(row0, rows_sub), :])

    out_type = [jax.ShapeDtypeStruct((_TBL, width), _F32),
                jax.ShapeDtypeStruct((_TBL, width), _F32)]
    scratch = [pltpu.VMEM((idx_sub, 128), jnp.int32),
               pltpu.VMEM((idx_sub, 128), jnp.int32)]
    if with_cnt:
        out_type += [jax.ShapeDtypeStruct((_TBL, 16), _F32),
                     jax.ShapeDtypeStruct((_TBL, 16), _F32)]
        scratch += [pltpu.VMEM((_NBUF, 128), jnp.int32)]
    scratch += [pltpu.VMEM((_NBUF, 128, width), _F32)]
    if with_cnt:
        scratch += [pltpu.VMEM((128, 16), _F32)]
    scratch += [pltpu.VMEM((16, width), _F32)]
    if with_cnt:
        scratch += [pltpu.VMEM((16, 16), _F32)]
    scratch += [pltpu.VMEM_SHARED((_TBL, width), _F32)]
    if with_cnt:
        scratch += [pltpu.VMEM_SHARED((_TBL, 16), _F32)]
    scratch += [pltpu.SemaphoreType.DMA] * (2 * _NBUF)

    mesh = plsc.VectorSubcoreMesh(core_axis_name="c", subcore_axis_name="s")
    return pl.kernel(body, out_type=out_type, mesh=mesh,
                     scratch_types=scratch,
                     compiler_params=pltpu.CompilerParams(
                         use_tc_tiling_on_sc=False),
                     name=f"sc_segsum_w{width}")


_sc_agg64 = _make_sc_agg(64, with_cnt=True)
_sc_agg32 = _make_sc_agg(32, with_cnt=False)


def _tc1_body(alo, ahi, ca, cb, x, wl1, wr1, b1, wl2, wr2, b2,
              plo, phi, r2o):
    cnt = ca[:, 0:1] + cb[:, 0:1]
    icnt = 1.0 / jnp.maximum(cnt, 1.0)
    h = (jnp.dot(alo[...] * icnt, wl1[0:64, :], preferred_element_type=_F32)
         + jnp.dot(ahi[...] * icnt, wl1[64:128, :], preferred_element_type=_F32)
         + jnp.dot(x[...], wr1[...], preferred_element_type=_F32)
         + b1[...])
    h = jnp.maximum(h, 0.0)
    p = jnp.dot(h, wl2[...], preferred_element_type=_F32)
    plo[...] = p[:, 0:32]
    phi[...] = p[:, 32:64]
    r2o[...] = jnp.dot(h, wr2[...], preferred_element_type=_F32) + b2[...]


def _tc2_body(a2lo, a2hi, ca, cb, r2, out):
    cnt = ca[:, 0:1] + cb[:, 0:1]
    icnt = 1.0 / jnp.maximum(cnt, 1.0)
    out[...] = (jnp.concatenate([a2lo[...] * icnt, a2hi[...] * icnt], axis=1)
                + r2[...])


_BLK = 1000


def _row(i):
    return (i, 0)


def _full(i):
    return (0, 0)


def _tc1(alo, ahi, ca, cb, x, wl1, wr1, b1, wl2, wr2, b2):
    return pl.pallas_call(
        _tc1_body,
        grid=(_N // _BLK,),
        in_specs=[
            pl.BlockSpec((_BLK, 64), _row),
            pl.BlockSpec((_BLK, 64), _row),
            pl.BlockSpec((_BLK, 16), _row),
            pl.BlockSpec((_BLK, 16), _row),
            pl.BlockSpec((_BLK, 128), _row),
            pl.BlockSpec((128, 128), _full),
            pl.BlockSpec((128, 128), _full),
            pl.BlockSpec((1, 128), _full),
            pl.BlockSpec((128, 64), _full),
            pl.BlockSpec((128, 64), _full),
            pl.BlockSpec((1, 64), _full),
        ],
        out_specs=[
            pl.BlockSpec((_BLK, 32), _row),
            pl.BlockSpec((_BLK, 32), _row),
            pl.BlockSpec((_BLK, 64), _row),
        ],
        out_shape=[
            jax.ShapeDtypeStruct((_N, 32), _F32),
            jax.ShapeDtypeStruct((_N, 32), _F32),
            jax.ShapeDtypeStruct((_N, 64), _F32),
        ],
    )(alo, ahi, ca, cb, x, wl1, wr1, b1, wl2, wr2, b2)


def _tc2(a2lo, a2hi, ca, cb, r2):
    return pl.pallas_call(
        _tc2_body,
        grid=(_N // _BLK,),
        in_specs=[
            pl.BlockSpec((_BLK, 32), _row),
            pl.BlockSpec((_BLK, 32), _row),
            pl.BlockSpec((_BLK, 16), _row),
            pl.BlockSpec((_BLK, 16), _row),
            pl.BlockSpec((_BLK, 64), _row),
        ],
        out_specs=pl.BlockSpec((_BLK, 64), _row),
        out_shape=jax.ShapeDtypeStruct((_N, 64), _F32),
    )(a2lo, a2hi, ca, cb, r2)


def kernel(x, edge_index, W_l1, W_r1, b1, W_l2, W_r2, b2):
    src = edge_index[0].astype(jnp.int32)
    dst = edge_index[1].astype(jnp.int32)
    pad = _EP - _E
    srcp = jnp.concatenate([src, jnp.zeros((pad,), jnp.int32)]).reshape(
        _IDX_ROWS, 128)
    dstp = jnp.concatenate([dst, jnp.full((pad,), _PAD_DST, jnp.int32)]
                           ).reshape(_IDX_ROWS, 128)
    ones = jnp.ones((128, 16), _F32)
    x_lo = x[:, 0:64]
    x_hi = x[:, 64:128]

    agg_lo, agg_hi, cnt_a, cnt_b = _sc_agg64(x_lo, x_hi, srcp, dstp, ones)
    p_lo, p_hi, r2 = _tc1(agg_lo, agg_hi, cnt_a, cnt_b, x,
                          W_l1, W_r1, b1.reshape(1, -1),
                          W_l2, W_r2, b2.reshape(1, -1))
    a2_lo, a2_hi = _sc_agg32(p_lo, p_hi, srcp, dstp)
    return _tc2(a2_lo, a2_hi, cnt_a, cnt_b, r2)


# speedup vs baseline: 6.0480x; 1.0850x over previous
import functools

import jax
import jax.numpy as jnp
from jax import lax
from jax.experimental import pallas as pl
from jax.experimental.pallas import tpu as pltpu
from jax.experimental.pallas import tpu_sc as plsc

_N = 10000
_E = 320000
_IDX_ROWS = 2560
_EP = _IDX_ROWS * 128
_TBL = 10240
_PAD_DST = _N
_NS = 16
_NBUF = 4
_F32 = jnp.float32


def _make_sc_agg(width, with_cnt):
    rows_sub = _TBL // _NS
    idx_sub = _IDX_ROWS // _NS
    cnt_sub = _IDX_ROWS // (2 * _NS)

    def body(*refs):
        if with_cnt:
            (xlo, xhi, srcp, dstp, ones_h,
             olo, ohi, cnt_a, cnt_b,
             src_v, dst_v, dstc_v, rows_v, ones_v, zbuf, zbuf_c,
             agg_sp, cnt_sp, *sems) = refs
        else:
            (xlo, xhi, srcp, dstp,
             olo, ohi,
             src_v, dst_v, rows_v, zbuf,
             agg_sp, *sems) = refs
        gsem = sems[:_NBUF]
        ssem = sems[_NBUF:]
        c = lax.axis_index("c")
        s = lax.axis_index("s")

        for r in range(16):
            for cc in range(width // 16):
                zbuf[r, pl.ds(cc * 16, 16)] = jnp.zeros((16,), _F32)
        row0 = s * rows_sub

        @pl.loop(0, rows_sub // 16)
        def _(k):
            pltpu.sync_copy(zbuf, agg_sp.at[pl.ds(row0 + k * 16, 16), :])

        if with_cnt:
            for r in range(16):
                zbuf_c[r, :] = jnp.zeros((16,), _F32)

            @pl.loop(0, rows_sub // 16)
            def _(k):
                pltpu.sync_copy(zbuf_c, cnt_sp.at[pl.ds(row0 + k * 16, 16), :])

            pltpu.sync_copy(ones_h, ones_v)

        pltpu.sync_copy(srcp.at[pl.ds(s * idx_sub, idx_sub), :], src_v)
        pltpu.sync_copy(dstp.at[pl.ds(s * idx_sub, idx_sub), :], dst_v)
        plsc.subcore_barrier()

        nb = idx_sub // _NBUF

        def edge_loop(x_hbm):
            for b in range(_NBUF):
                pltpu.async_copy(x_hbm.at[src_v.at[b]], rows_v.at[b],
                                 gsem[b])

            @pl.loop(0, nb)
            def _(t):
                for b in range(_NBUF):
                    j = t * _NBUF + b
                    pltpu.make_async_copy(
                        x_hbm.at[src_v.at[j]], rows_v.at[b], gsem[b]).wait()
                    pltpu.async_copy(rows_v.at[b], agg_sp.at[dst_v.at[j]],
                                     ssem[b], add=True)

                @pl.when(t < nb - 1)
                def _():
                    for b in range(_NBUF):
                        j = t * _NBUF + b
                        pltpu.make_async_copy(
                            rows_v.at[b], agg_sp.at[dst_v.at[j]],
                            ssem[b]).wait()
                        pltpu.async_copy(x_hbm.at[src_v.at[j + _NBUF]],
                                         rows_v.at[b], gsem[b])

            for b in range(_NBUF):
                j = (nb - 1) * _NBUF + b
                pltpu.make_async_copy(rows_v.at[b], agg_sp.at[dst_v.at[j]],
                                      ssem[b]).wait()

        @pl.when(c == 0)
        def _():
            edge_loop(xlo)

        @pl.when(c == 1)
        def _():
            edge_loop(xhi)

        if with_cnt:
            cbase = c * (_IDX_ROWS // 2) + s * cnt_sub

            @pl.loop(0, cnt_sub // _NBUF)
            def _(t):
                pltpu.sync_copy(
                    dstp.at[pl.ds(cbase + t * _NBUF, _NBUF), :], dstc_v)
                sds = []
                for b in range(_NBUF):
                    sds.append(pltpu.async_copy(
                        ones_v, cnt_sp.at[dstc_v.at[b]], ssem[b], add=True))
                for sd in sds:
                    sd.wait()

        plsc.subcore_barrier()

        @pl.when(c == 0)
        def _():
            pltpu.sync_copy(agg_sp.at[pl.ds(row0, rows_sub), :],
                            olo.at[pl.ds(row0, rows_sub), :])
            if with_cnt:
                pltpu.sync_copy(cnt_sp.at[pl.ds(row0, rows_sub), :],
                                cnt_a.at[pl.ds(row0, rows_sub), :])

        @pl.when(c == 1)
        def _():
            pltpu.sync_copy(agg_sp.at[pl.ds(row0, rows_sub), :],
                            ohi.at[pl.ds(row0, rows_sub), :])
            if with_cnt:
                pltpu.sync_copy(cnt_sp.at[pl.ds(row0, rows_sub), :],
                                cnt_b.at[pl.ds(row0, rows_sub), :])

    out_type = [jax.ShapeDtypeStruct((_TBL, width), _F32),
                jax.ShapeDtypeStruct((_TBL, width), _F32)]
    scratch = [pltpu.VMEM((idx_sub, 128), jnp.int32),
               pltpu.VMEM((idx_sub, 128), jnp.int32)]
    if with_cnt:
        out_type += [jax.ShapeDtypeStruct((_TBL, 16), _F32),
                     jax.ShapeDtypeStruct((_TBL, 16), _F32)]
        scratch += [pltpu.VMEM((_NBUF, 128), jnp.int32)]
    scratch += [pltpu.VMEM((_NBUF, 128, width), _F32)]
    if with_cnt:
        scratch += [pltpu.VMEM((128, 16), _F32)]
    scratch += [pltpu.VMEM((16, width), _F32)]
    if with_cnt:
        scratch += [pltpu.VMEM((16, 16), _F32)]
    scratch += [pltpu.VMEM_SHARED((_TBL, width), _F32)]
    if with_cnt:
        scratch += [pltpu.VMEM_SHARED((_TBL, 16), _F32)]
    scratch += [pltpu.SemaphoreType.DMA] * (2 * _NBUF)

    mesh = plsc.VectorSubcoreMesh(core_axis_name="c", subcore_axis_name="s")
    return pl.kernel(body, out_type=out_type, mesh=mesh,
                     scratch_types=scratch,
                     compiler_params=pltpu.CompilerParams(
                         use_tc_tiling_on_sc=False),
                     name=f"sc_segsum_w{width}")


_sc_agg64 = _make_sc_agg(64, with_cnt=True)
_sc_agg32 = _make_sc_agg(32, with_cnt=False)


def _tc1_body(alo, ahi, ca, cb, x, wl1, wr1, b1, wl2, wr2, b2,
              plo, phi, r2o):
    cnt = ca[:, 0:1] + cb[:, 0:1]
    icnt = 1.0 / jnp.maximum(cnt, 1.0)
    h = (jnp.dot(alo[...] * icnt, wl1[0:64, :], preferred_element_type=_F32)
         + jnp.dot(ahi[...] * icnt, wl1[64:128, :], preferred_element_type=_F32)
         + jnp.dot(x[...], wr1[...], preferred_element_type=_F32)
         + b1[...])
    h = jnp.maximum(h, 0.0)
    p = jnp.dot(h, wl2[...], preferred_element_type=_F32)
    plo[...] = p[:, 0:32]
    phi[...] = p[:, 32:64]
    r2o[...] = jnp.dot(h, wr2[...], preferred_element_type=_F32) + b2[...]


def _tc2_body(a2lo, a2hi, ca, cb, r2, out):
    cnt = ca[:, 0:1] + cb[:, 0:1]
    icnt = 1.0 / jnp.maximum(cnt, 1.0)
    out[...] = (jnp.concatenate([a2lo[...] * icnt, a2hi[...] * icnt], axis=1)
                + r2[...])


_BLK = 1000


def _row(i):
    return (i, 0)


def _full(i):
    return (0, 0)


def _tc1(alo, ahi, ca, cb, x, wl1, wr1, b1, wl2, wr2, b2):
    return pl.pallas_call(
        _tc1_body,
        grid=(_N // _BLK,),
        in_specs=[
            pl.BlockSpec((_BLK, 64), _row),
            pl.BlockSpec((_BLK, 64), _row),
            pl.BlockSpec((_BLK, 16), _row),
            pl.BlockSpec((_BLK, 16), _row),
            pl.BlockSpec((_BLK, 128), _row),
            pl.BlockSpec((128, 128), _full),
            pl.BlockSpec((128, 128), _full),
            pl.BlockSpec((1, 128), _full),
            pl.BlockSpec((128, 64), _full),
            pl.BlockSpec((128, 64), _full),
            pl.BlockSpec((1, 64), _full),
        ],
        out_specs=[
            pl.BlockSpec((_BLK, 32), _row),
            pl.BlockSpec((_BLK, 32), _row),
            pl.BlockSpec((_BLK, 64), _row),
        ],
        out_shape=[
            jax.ShapeDtypeStruct((_N, 32), _F32),
            jax.ShapeDtypeStruct((_N, 32), _F32),
            jax.ShapeDtypeStruct((_N, 64), _F32),
        ],
    )(alo, ahi, ca, cb, x, wl1, wr1, b1, wl2, wr2, b2)


def _tc2(a2lo, a2hi, ca, cb, r2):
    return pl.pallas_call(
        _tc2_body,
        grid=(_N // _BLK,),
        in_specs=[
            pl.BlockSpec((_BLK, 32), _row),
            pl.BlockSpec((_BLK, 32), _row),
            pl.BlockSpec((_BLK, 16), _row),
            pl.BlockSpec((_BLK, 16), _row),
            pl.BlockSpec((_BLK, 64), _row),
        ],
        out_specs=pl.BlockSpec((_BLK, 64), _row),
        out_shape=jax.ShapeDtypeStruct((_N, 64), _F32),
    )(a2lo, a2hi, ca, cb, r2)


def kernel(x, edge_index, W_l1, W_r1, b1, W_l2, W_r2, b2):
    src = edge_index[0].astype(jnp.int32)
    dst = edge_index[1].astype(jnp.int32)
    pad = _EP - _E
    srcp = jnp.concatenate([src, jnp.zeros((pad,), jnp.int32)]).reshape(
        _IDX_ROWS, 128)
    dstp = jnp.concatenate([dst, jnp.full((pad,), _PAD_DST, jnp.int32)]
                           ).reshape(_IDX_ROWS, 128)
    ones = jnp.ones((128, 16), _F32)
    x_lo = x[:, 0:64]
    x_hi = x[:, 64:128]

    agg_lo, agg_hi, cnt_a, cnt_b = _sc_agg64(x_lo, x_hi, srcp, dstp, ones)
    p_lo, p_hi, r2 = _tc1(agg_lo, agg_hi, cnt_a, cnt_b, x,
                          W_l1, W_r1, b1.reshape(1, -1),
                          W_l2, W_r2, b2.reshape(1, -1))
    a2_lo, a2_hi = _sc_agg32(p_lo, p_hi, srcp, dstp)
    return _tc2(a2_lo, a2_hi, cnt_a, cnt_b, r2)


# speedup vs baseline: 6.2085x; 1.0265x over previous
import functools

import jax
import jax.numpy as jnp
from jax import lax
from jax.experimental import pallas as pl
from jax.experimental.pallas import tpu as pltpu
from jax.experimental.pallas import tpu_sc as plsc

_N = 10000
_E = 320000
_IDX_ROWS = 2560
_EP = _IDX_ROWS * 128
_TBL = 10240
_PAD_DST = _N
_NS = 16
_NBUF = 8
_F32 = jnp.float32


def _make_sc_agg(width, with_cnt):
    rows_sub = _TBL // _NS
    idx_sub = _IDX_ROWS // _NS
    cnt_sub = _IDX_ROWS // (2 * _NS)

    def body(*refs):
        if with_cnt:
            (xlo, xhi, srcp, dstp, ones_h,
             olo, ohi, cnt_a, cnt_b,
             sbuf, dbuf, dstc_v, rows_v, ones_v, zbuf, zbuf_c,
             agg_sp, cnt_sp, *sems) = refs
        else:
            (xlo, xhi, srcp, dstp,
             olo, ohi,
             sbuf, dbuf, rows_v, zbuf,
             agg_sp, *sems) = refs
        gsem = sems[:_NBUF]
        ssem = sems[_NBUF:2 * _NBUF]
        isem = sems[2 * _NBUF]
        c = lax.axis_index("c")
        s = lax.axis_index("s")

        for r in range(16):
            for cc in range(width // 16):
                zbuf[r, pl.ds(cc * 16, 16)] = jnp.zeros((16,), _F32)
        row0 = s * rows_sub

        @pl.loop(0, rows_sub // 16)
        def _(k):
            pltpu.sync_copy(zbuf, agg_sp.at[pl.ds(row0 + k * 16, 16), :])

        if with_cnt:
            for r in range(16):
                zbuf_c[r, :] = jnp.zeros((16,), _F32)

            @pl.loop(0, rows_sub // 16)
            def _(k):
                pltpu.sync_copy(zbuf_c, cnt_sp.at[pl.ds(row0 + k * 16, 16), :])

            pltpu.sync_copy(ones_h, ones_v)

        plsc.subcore_barrier()

        nb = idx_sub // _NBUF
        base = s * idx_sub

        def edge_loop(x_hbm):
            pltpu.sync_copy(srcp.at[pl.ds(base, _NBUF), :], sbuf.at[0])
            pltpu.sync_copy(dstp.at[pl.ds(base, _NBUF), :], dbuf.at[0])
            for b in range(_NBUF):
                pltpu.async_copy(x_hbm.at[sbuf.at[0, b]], rows_v.at[b],
                                 gsem[b])

            def batch(t, u, guard):
                q = 1 - u
                off = base + (2 * t + u + 1) * _NBUF

                def fire_idx():
                    pltpu.async_copy(srcp.at[pl.ds(off, _NBUF), :],
                                     sbuf.at[q], isem)
                    pltpu.async_copy(dstp.at[pl.ds(off, _NBUF), :],
                                     dbuf.at[q], isem)

                if guard is None:
                    fire_idx()
                else:
                    @pl.when(guard)
                    def _():
                        fire_idx()

                for b in range(_NBUF):
                    pltpu.make_async_copy(x_hbm.at[sbuf.at[u, b]],
                                          rows_v.at[b], gsem[b]).wait()
                    pltpu.async_copy(rows_v.at[b],
                                     agg_sp.at[dbuf.at[u, b]],
                                     ssem[b], add=True)

                def next_gathers():
                    pltpu.make_async_copy(srcp.at[pl.ds(off, _NBUF), :],
                                          sbuf.at[q], isem).wait()
                    pltpu.make_async_copy(dstp.at[pl.ds(off, _NBUF), :],
                                          dbuf.at[q], isem).wait()
                    for b in range(_NBUF):
                        pltpu.make_async_copy(rows_v.at[b],
                                              agg_sp.at[dbuf.at[u, b]],
                                              ssem[b]).wait()
                        pltpu.async_copy(x_hbm.at[sbuf.at[q, b]],
                                         rows_v.at[b], gsem[b])

                if guard is None:
                    next_gathers()
                else:
                    @pl.when(guard)
                    def _():
                        next_gathers()

            @pl.loop(0, nb // 2)
            def _(t):
                batch(t, 0, None)
                batch(t, 1, t < nb // 2 - 1)

            p_last = (nb - 1) % 2
            for b in range(_NBUF):
                pltpu.make_async_copy(rows_v.at[b],
                                      agg_sp.at[dbuf.at[p_last, b]],
                                      ssem[b]).wait()

        @pl.when(c == 0)
        def _():
            edge_loop(xlo)

        @pl.when(c == 1)
        def _():
            edge_loop(xhi)

        if with_cnt:
            cbase = c * (_IDX_ROWS // 2) + s * cnt_sub

            @pl.loop(0, cnt_sub // _NBUF)
            def _(t):
                pltpu.sync_copy(
                    dstp.at[pl.ds(cbase + t * _NBUF, _NBUF), :], dstc_v)
                sds = []
                for b in range(_NBUF):
                    sds.append(pltpu.async_copy(
                        ones_v, cnt_sp.at[dstc_v.at[b]], ssem[b], add=True))
                for sd in sds:
                    sd.wait()

        plsc.subcore_barrier()

        @pl.when(c == 0)
        def _():
            pltpu.sync_copy(agg_sp.at[pl.ds(row0, rows_sub), :],
                            olo.at[pl.ds(row0, rows_sub), :])
            if with_cnt:
                pltpu.sync_copy(cnt_sp.at[pl.ds(row0, rows_sub), :],
                                cnt_a.at[pl.ds(row0, rows_sub), :])

        @pl.when(c == 1)
        def _():
            pltpu.sync_copy(agg_sp.at[pl.ds(row0, rows_sub), :],
                            ohi.at[pl.ds(row0, rows_sub), :])
            if with_cnt:
                pltpu.sync_copy(cnt_sp.at[pl.ds(row0, rows_sub), :],
                                cnt_b.at[pl.ds(row0, rows_sub), :])

    out_type = [jax.ShapeDtypeStruct((_TBL, width), _F32),
                jax.ShapeDtypeStruct((_TBL, width), _F32)]
    scratch = [pltpu.VMEM((2, _NBUF, 128), jnp.int32),
               pltpu.VMEM((2, _NBUF, 128), jnp.int32)]
    if with_cnt:
        out_type += [jax.ShapeDtypeStruct((_TBL, 16), _F32),
                     jax.ShapeDtypeStruct((_TBL, 16), _F32)]
        scratch += [pltpu.VMEM((_NBUF, 128), jnp.int32)]
    scratch += [pltpu.VMEM((_NBUF, 128, width), _F32)]
    if with_cnt:
        scratch += [pltpu.VMEM((128, 16), _F32)]
    scratch += [pltpu.VMEM((16, width), _F32)]
    if with_cnt:
        scratch += [pltpu.VMEM((16, 16), _F32)]
    scratch += [pltpu.VMEM_SHARED((_TBL, width), _F32)]
    if with_cnt:
        scratch += [pltpu.VMEM_SHARED((_TBL, 16), _F32)]
    scratch += [pltpu.SemaphoreType.DMA] * (2 * _NBUF + 1)

    mesh = plsc.VectorSubcoreMesh(core_axis_name="c", subcore_axis_name="s")
    return pl.kernel(body, out_type=out_type, mesh=mesh,
                     scratch_types=scratch,
                     compiler_params=pltpu.CompilerParams(
                         use_tc_tiling_on_sc=False),
                     name=f"sc_segsum_w{width}")


_sc_agg64 = _make_sc_agg(64, with_cnt=True)
_sc_agg32 = _make_sc_agg(32, with_cnt=False)


def _tc1_body(alo, ahi, ca, cb, x, wl1, wr1, b1, wl2, wr2, b2,
              plo, phi, r2o):
    cnt = ca[:, 0:1] + cb[:, 0:1]
    icnt = 1.0 / jnp.maximum(cnt, 1.0)
    h = (jnp.dot(alo[...] * icnt, wl1[0:64, :], preferred_element_type=_F32)
         + jnp.dot(ahi[...] * icnt, wl1[64:128, :], preferred_element_type=_F32)
         + jnp.dot(x[...], wr1[...], preferred_element_type=_F32)
         + b1[...])
    h = jnp.maximum(h, 0.0)
    p = jnp.dot(h, wl2[...], preferred_element_type=_F32)
    plo[...] = p[:, 0:32]
    phi[...] = p[:, 32:64]
    r2o[...] = jnp.dot(h, wr2[...], preferred_element_type=_F32) + b2[...]


def _tc2_body(a2lo, a2hi, ca, cb, r2, out):
    cnt = ca[:, 0:1] + cb[:, 0:1]
    icnt = 1.0 / jnp.maximum(cnt, 1.0)
    out[...] = (jnp.concatenate([a2lo[...] * icnt, a2hi[...] * icnt], axis=1)
                + r2[...])


_BLK = 1000


def _row(i):
    return (i, 0)


def _full(i):
    return (0, 0)


def _tc1(alo, ahi, ca, cb, x, wl1, wr1, b1, wl2, wr2, b2):
    return pl.pallas_call(
        _tc1_body,
        grid=(_N // _BLK,),
        in_specs=[
            pl.BlockSpec((_BLK, 64), _row),
            pl.BlockSpec((_BLK, 64), _row),
            pl.BlockSpec((_BLK, 16), _row),
            pl.BlockSpec((_BLK, 16), _row),
            pl.BlockSpec((_BLK, 128), _row),
            pl.BlockSpec((128, 128), _full),
            pl.BlockSpec((128, 128), _full),
            pl.BlockSpec((1, 128), _full),
            pl.BlockSpec((128, 64), _full),
            pl.BlockSpec((128, 64), _full),
            pl.BlockSpec((1, 64), _full),
        ],
        out_specs=[
            pl.BlockSpec((_BLK, 32), _row),
            pl.BlockSpec((_BLK, 32), _row),
            pl.BlockSpec((_BLK, 64), _row),
        ],
        out_shape=[
            jax.ShapeDtypeStruct((_N, 32), _F32),
            jax.ShapeDtypeStruct((_N, 32), _F32),
            jax.ShapeDtypeStruct((_N, 64), _F32),
        ],
    )(alo, ahi, ca, cb, x, wl1, wr1, b1, wl2, wr2, b2)


def _tc2(a2lo, a2hi, ca, cb, r2):
    return pl.pallas_call(
        _tc2_body,
        grid=(_N // _BLK,),
        in_specs=[
            pl.BlockSpec((_BLK, 32), _row),
            pl.BlockSpec((_BLK, 32), _row),
            pl.BlockSpec((_BLK, 16), _row),
            pl.BlockSpec((_BLK, 16), _row),
            pl.BlockSpec((_BLK, 64), _row),
        ],
        out_specs=pl.BlockSpec((_BLK, 64), _row),
        out_shape=jax.ShapeDtypeStruct((_N, 64), _F32),
    )(a2lo, a2hi, ca, cb, r2)


def kernel(x, edge_index, W_l1, W_r1, b1, W_l2, W_r2, b2):
    src = edge_index[0].astype(jnp.int32)
    dst = edge_index[1].astype(jnp.int32)
    pad = _EP - _E
    srcp = jnp.concatenate([src, jnp.zeros((pad,), jnp.int32)]).reshape(
        _IDX_ROWS, 128)
    dstp = jnp.concatenate([dst, jnp.full((pad,), _PAD_DST, jnp.int32)]
                           ).reshape(_IDX_ROWS, 128)
    ones = jnp.ones((128, 16), _F32)
    x_lo = x[:, 0:64]
    x_hi = x[:, 64:128]

    agg_lo, agg_hi, cnt_a, cnt_b = _sc_agg64(x_lo, x_hi, srcp, dstp, ones)
    p_lo, p_hi, r2 = _tc1(agg_lo, agg_hi, cnt_a, cnt_b, x,
                          W_l1, W_r1, b1.reshape(1, -1),
                          W_l2, W_r2, b2.reshape(1, -1))
    a2_lo, a2_hi = _sc_agg32(p_lo, p_hi, srcp, dstp)
    return _tc2(a2_lo, a2_hi, cnt_a, cnt_b, r2)


# speedup vs baseline: 6.3101x; 1.0164x over previous
import functools

import jax
import jax.numpy as jnp
from jax import lax
from jax.experimental import pallas as pl
from jax.experimental.pallas import tpu as pltpu
from jax.experimental.pallas import tpu_sc as plsc

_N = 10000
_E = 320000
_IDX_ROWS = 2560
_EP = _IDX_ROWS * 128
_TBL = 10240
_PAD_DST = _N
_NS = 16
_NBUF = 8
_CNB = 4
_F32 = jnp.float32


def _make_sc_agg(width, with_cnt):
    rows_sub = _TBL // _NS
    idx_sub = _IDX_ROWS // _NS
    cnt_sub = _IDX_ROWS // (2 * _NS)

    def body(*refs):
        if with_cnt:
            (xlo, xhi, srcp, dstp, ones_h,
             olo, ohi, cnt_a, cnt_b,
             sbuf, dbuf, dstc_v, rows_v, ones_v, zbuf, zbuf_c,
             agg_sp, cnt_sp, *sems) = refs
        else:
            (xlo, xhi, srcp, dstp,
             olo, ohi,
             sbuf, dbuf, rows_v, zbuf,
             agg_sp, *sems) = refs
        gsem = sems[:_NBUF]
        ssem = sems[_NBUF:2 * _NBUF]
        isem = sems[2 * _NBUF]
        if with_cnt:
            cisem = sems[2 * _NBUF + 1]
            csem = sems[2 * _NBUF + 2:]
        c = lax.axis_index("c")
        s = lax.axis_index("s")

        for r in range(16):
            for cc in range(width // 16):
                zbuf[r, pl.ds(cc * 16, 16)] = jnp.zeros((16,), _F32)
        row0 = s * rows_sub

        @pl.loop(0, rows_sub // 16)
        def _(k):
            pltpu.sync_copy(zbuf, agg_sp.at[pl.ds(row0 + k * 16, 16), :])

        if with_cnt:
            for r in range(16):
                zbuf_c[r, :] = jnp.zeros((16,), _F32)

            @pl.loop(0, rows_sub // 16)
            def _(k):
                pltpu.sync_copy(zbuf_c, cnt_sp.at[pl.ds(row0 + k * 16, 16), :])

            pltpu.sync_copy(ones_h, ones_v)

        plsc.subcore_barrier()

        nb = idx_sub // _NBUF
        base = s * idx_sub
        cbase = c * (_IDX_ROWS // 2) + s * cnt_sub

        def edge_loop(x_hbm):
            pltpu.sync_copy(srcp.at[pl.ds(base, _NBUF), :], sbuf.at[0])
            pltpu.sync_copy(dstp.at[pl.ds(base, _NBUF), :], dbuf.at[0])
            if with_cnt:
                pltpu.sync_copy(dstp.at[pl.ds(cbase, _CNB), :], dstc_v.at[0])
            for b in range(_NBUF):
                pltpu.async_copy(x_hbm.at[sbuf.at[0, b]], rows_v.at[b],
                                 gsem[b])

            def batch(t, u, guard):
                q = 1 - u
                off = base + (2 * t + u + 1) * _NBUF

                if with_cnt:
                    coff_next = cbase + (2 * t + u + 1) * _CNB
                    coff_cur = cbase + (2 * t + u) * _CNB

                    def cnt_wait_prev():
                        for b in range(_CNB):
                            pltpu.make_async_copy(
                                ones_v, cnt_sp.at[dstc_v.at[q, b]],
                                csem[b]).wait()

                    def cnt_idx_wait():
                        pltpu.make_async_copy(
                            dstp.at[pl.ds(coff_cur, _CNB), :],
                            dstc_v.at[u], cisem).wait()

                    if u == 0:
                        @pl.when(t > 0)
                        def _():
                            cnt_wait_prev()
                            cnt_idx_wait()
                    else:
                        cnt_wait_prev()
                        cnt_idx_wait()

                    def cnt_idx_fire():
                        pltpu.async_copy(dstp.at[pl.ds(coff_next, _CNB), :],
                                         dstc_v.at[q], cisem)

                    if guard is None:
                        cnt_idx_fire()
                    else:
                        @pl.when(guard)
                        def _():
                            cnt_idx_fire()

                    for b in range(_CNB):
                        pltpu.async_copy(ones_v,
                                         cnt_sp.at[dstc_v.at[u, b]],
                                         csem[b], add=True)

                def fire_idx():
                    pltpu.async_copy(srcp.at[pl.ds(off, _NBUF), :],
                                     sbuf.at[q], isem)
                    pltpu.async_copy(dstp.at[pl.ds(off, _NBUF), :],
                                     dbuf.at[q], isem)

                if guard is None:
                    fire_idx()
                else:
                    @pl.when(guard)
                    def _():
                        fire_idx()

                for b in range(_NBUF):
                    pltpu.make_async_copy(x_hbm.at[sbuf.at[u, b]],
                                          rows_v.at[b], gsem[b]).wait()
                    pltpu.async_copy(rows_v.at[b],
                                     agg_sp.at[dbuf.at[u, b]],
                                     ssem[b], add=True)

                def next_gathers():
                    pltpu.make_async_copy(srcp.at[pl.ds(off, _NBUF), :],
                                          sbuf.at[q], isem).wait()
                    pltpu.make_async_copy(dstp.at[pl.ds(off, _NBUF), :],
                                          dbuf.at[q], isem).wait()
                    for b in range(_NBUF):
                        pltpu.make_async_copy(rows_v.at[b],
                                              agg_sp.at[dbuf.at[u, b]],
                                              ssem[b]).wait()
                        pltpu.async_copy(x_hbm.at[sbuf.at[q, b]],
                                         rows_v.at[b], gsem[b])

                if guard is None:
                    next_gathers()
                else:
                    @pl.when(guard)
                    def _():
                        next_gathers()

            @pl.loop(0, nb // 2)
            def _(t):
                batch(t, 0, None)
                batch(t, 1, t < nb // 2 - 1)

            p_last = (nb - 1) % 2
            for b in range(_NBUF):
                pltpu.make_async_copy(rows_v.at[b],
                                      agg_sp.at[dbuf.at[p_last, b]],
                                      ssem[b]).wait()
            if with_cnt:
                for b in range(_CNB):
                    pltpu.make_async_copy(ones_v,
                                          cnt_sp.at[dstc_v.at[p_last, b]],
                                          csem[b]).wait()

        @pl.when(c == 0)
        def _():
            edge_loop(xlo)

        @pl.when(c == 1)
        def _():
            edge_loop(xhi)

        plsc.subcore_barrier()

        @pl.when(c == 0)
        def _():
            pltpu.sync_copy(agg_sp.at[pl.ds(row0, rows_sub), :],
                            olo.at[pl.ds(row0, rows_sub), :])
            if with_cnt:
                pltpu.sync_copy(cnt_sp.at[pl.ds(row0, rows_sub), :],
                                cnt_a.at[pl.ds(row0, rows_sub), :])

        @pl.when(c == 1)
        def _():
            pltpu.sync_copy(agg_sp.at[pl.ds(row0, rows_sub), :],
                            ohi.at[pl.ds(row0, rows_sub), :])
            if with_cnt:
                pltpu.sync_copy(cnt_sp.at[pl.ds(row0, rows_sub), :],
                                cnt_b.at[pl.ds(row0, rows_sub), :])

    out_type = [jax.ShapeDtypeStruct((_TBL, width), _F32),
                jax.ShapeDtypeStruct((_TBL, width), _F32)]
    scratch = [pltpu.VMEM((2, _NBUF, 128), jnp.int32),
               pltpu.VMEM((2, _NBUF, 128), jnp.int32)]
    if with_cnt:
        out_type += [jax.ShapeDtypeStruct((_TBL, 16), _F32),
                     jax.ShapeDtypeStruct((_TBL, 16), _F32)]
        scratch += [pltpu.VMEM((2, _CNB, 128), jnp.int32)]
    scratch += [pltpu.VMEM((_NBUF, 128, width), _F32)]
    if with_cnt:
        scratch += [pltpu.VMEM((128, 16), _F32)]
    scratch += [pltpu.VMEM((16, width), _F32)]
    if with_cnt:
        scratch += [pltpu.VMEM((16, 16), _F32)]
    scratch += [pltpu.VMEM_SHARED((_TBL, width), _F32)]
    if with_cnt:
        scratch += [pltpu.VMEM_SHARED((_TBL, 16), _F32)]
    nsem = 2 * _NBUF + 1 + (1 + _CNB if with_cnt else 0)
    scratch += [pltpu.SemaphoreType.DMA] * nsem

    mesh = plsc.VectorSubcoreMesh(core_axis_name="c", subcore_axis_name="s")
    return pl.kernel(body, out_type=out_type, mesh=mesh,
                     scratch_types=scratch,
                     compiler_params=pltpu.CompilerParams(
                         use_tc_tiling_on_sc=False),
                     name=f"sc_segsum_w{width}")


_sc_agg64 = _make_sc_agg(64, with_cnt=True)
_sc_agg32 = _make_sc_agg(32, with_cnt=False)


def _tc1_body(alo, ahi, ca, cb, x, wl1, wr1, b1, wl2, wr2, b2,
              plo, phi, r2o):
    cnt = ca[:, 0:1] + cb[:, 0:1]
    icnt = 1.0 / jnp.maximum(cnt, 1.0)
    h = (jnp.dot(alo[...] * icnt, wl1[0:64, :], preferred_element_type=_F32)
         + jnp.dot(ahi[...] * icnt, wl1[64:128, :], preferred_element_type=_F32)
         + jnp.dot(x[...], wr1[...], preferred_element_type=_F32)
         + b1[...])
    h = jnp.maximum(h, 0.0)
    p = jnp.dot(h, wl2[...], preferred_element_type=_F32)
    plo[...] = p[:, 0:32]
    phi[...] = p[:, 32:64]
    r2o[...] = jnp.dot(h, wr2[...], preferred_element_type=_F32) + b2[...]


def _tc2_body(a2lo, a2hi, ca, cb, r2, out):
    cnt = ca[:, 0:1] + cb[:, 0:1]
    icnt = 1.0 / jnp.maximum(cnt, 1.0)
    out[...] = (jnp.concatenate([a2lo[...] * icnt, a2hi[...] * icnt], axis=1)
                + r2[...])


_BLK = 1000


def _row(i):
    return (i, 0)


def _full(i):
    return (0, 0)


def _tc1(alo, ahi, ca, cb, x, wl1, wr1, b1, wl2, wr2, b2):
    return pl.pallas_call(
        _tc1_body,
        grid=(_N // _BLK,),
        in_specs=[
            pl.BlockSpec((_BLK, 64), _row),
            pl.BlockSpec((_BLK, 64), _row),
            pl.BlockSpec((_BLK, 16), _row),
            pl.BlockSpec((_BLK, 16), _row),
            pl.BlockSpec((_BLK, 128), _row),
            pl.BlockSpec((128, 128), _full),
            pl.BlockSpec((128, 128), _full),
            pl.BlockSpec((1, 128), _full),
            pl.BlockSpec((128, 64), _full),
            pl.BlockSpec((128, 64), _full),
            pl.BlockSpec((1, 64), _full),
        ],
        out_specs=[
            pl.BlockSpec((_BLK, 32), _row),
            pl.BlockSpec((_BLK, 32), _row),
            pl.BlockSpec((_BLK, 64), _row),
        ],
        out_shape=[
            jax.ShapeDtypeStruct((_N, 32), _F32),
            jax.ShapeDtypeStruct((_N, 32), _F32),
            jax.ShapeDtypeStruct((_N, 64), _F32),
        ],
    )(alo, ahi, ca, cb, x, wl1, wr1, b1, wl2, wr2, b2)


def _tc2(a2lo, a2hi, ca, cb, r2):
    return pl.pallas_call(
        _tc2_body,
        grid=(_N // _BLK,),
        in_specs=[
            pl.BlockSpec((_BLK, 32), _row),
            pl.BlockSpec((_BLK, 32), _row),
            pl.BlockSpec((_BLK, 16), _row),
            pl.BlockSpec((_BLK, 16), _row),
            pl.BlockSpec((_BLK, 64), _row),
        ],
        out_specs=pl.BlockSpec((_BLK, 64), _row),
        out_shape=jax.ShapeDtypeStruct((_N, 64), _F32),
    )(a2lo, a2hi, ca, cb, r2)


def kernel(x, edge_index, W_l1, W_r1, b1, W_l2, W_r2, b2):
    src = edge_index[0].astype(jnp.int32)
    dst = edge_index[1].astype(jnp.int32)
    pad = _EP - _E
    srcp = jnp.concatenate([src, jnp.zeros((pad,), jnp.int32)]).reshape(
        _IDX_ROWS, 128)
    dstp = jnp.concatenate([dst, jnp.full((pad,), _PAD_DST, jnp.int32)]
                           ).reshape(_IDX_ROWS, 128)
    ones = jnp.ones((128, 16), _F32)
    x_lo = x[:, 0:64]
    x_hi = x[:, 64:128]

    agg_lo, agg_hi, cnt_a, cnt_b = _sc_agg64(x_lo, x_hi, srcp, dstp, ones)
    p_lo, p_hi, r2 = _tc1(agg_lo, agg_hi, cnt_a, cnt_b, x,
                          W_l1, W_r1, b1.reshape(1, -1),
                          W_l2, W_r2, b2.reshape(1, -1))
    a2_lo, a2_hi = _sc_agg32(p_lo, p_hi, srcp, dstp)
    return _tc2(a2_lo, a2_hi, cnt_a, cnt_b, r2)
